# baseline (speedup 1.0000x reference)
import functools

import jax
import jax.numpy as jnp
from jax.experimental import pallas as pl
from jax.experimental.pallas import tpu as pltpu

_F32 = jnp.float32


def _elu(v):
    return jnp.where(v > 0, v, jnp.exp(jnp.minimum(v, 0.0)) - 1.0)


def _hidden_body(ea_ref, w1_ref, b1_ref, w2_ref, b2_ref, w3_ref, b3_ref,
                 h1_ref, h2_ref, h3_ref):
    ea = ea_ref[...]
    h1_ref[...] = jnp.maximum(
        jnp.dot(ea, w1_ref[...], preferred_element_type=_F32) + b1_ref[...], 0.0)
    h2_ref[...] = jnp.maximum(
        jnp.dot(ea, w2_ref[...], preferred_element_type=_F32) + b2_ref[...], 0.0)
    h3_ref[...] = jnp.maximum(
        jnp.dot(ea, w3_ref[...], preferred_element_type=_F32) + b3_ref[...], 0.0)


def _layer1_body(x_ref, src_ref, dst_ref, h_ref, wb_ref, bbm_ref, root_ref,
                 bias_ref, out_ref, *, in_c, out_c):
    ee = src_ref.shape[0]
    nn = x_ref.shape[0]
    oh = (src_ref[...] == jax.lax.broadcasted_iota(jnp.int32, (ee, nn), 1)
          ).astype(_F32)
    xg = jnp.dot(oh, x_ref[...], preferred_element_type=_F32)
    acc = jnp.dot(xg, bbm_ref[...], preferred_element_type=_F32)
    h = h_ref[...]
    for i in range(in_c):
        t = jnp.dot(h, wb_ref[:, i * out_c:(i + 1) * out_c],
                    preferred_element_type=_F32)
        acc = acc + xg[:, i:i + 1] * t
    oht = (jax.lax.broadcasted_iota(jnp.int32, (nn, ee), 0) == dst_ref[...]
           ).astype(_F32)
    agg = jnp.dot(oht, acc, preferred_element_type=_F32)
    out_ref[...] = _elu(
        agg + jnp.dot(x_ref[...], root_ref[...], preferred_element_type=_F32)
        + bias_ref[...])


def _layer_body(x_ref, src_ref, dst_ref, h_ref, wb_ref, bbm_ref, root_ref,
                bias_ref, out_ref, xg_ref, acc_ref, *, ck, in_c):
    c = pl.program_id(0)
    ee = src_ref.shape[0]
    nn = x_ref.shape[0]

    @pl.when(c == 0)
    def _():
        oh = (src_ref[...] == jax.lax.broadcasted_iota(jnp.int32, (ee, nn), 1)
              ).astype(_F32)
        xg_ref[...] = jnp.dot(oh, x_ref[...], preferred_element_type=_F32)
        acc_ref[...] = jnp.dot(xg_ref[...], bbm_ref[...],
                               preferred_element_type=_F32)

    xg = xg_ref[...]
    hblk = h_ref[0]
    ee2 = xg.shape[0]
    z = (hblk[:, :, None] * xg[:, None, :]).reshape(ee2, ck * in_c)
    acc_ref[...] = acc_ref[...] + jnp.dot(
        z, wb_ref[...], preferred_element_type=_F32)

    @pl.when(c == pl.num_programs(0) - 1)
    def _():
        oht = (jax.lax.broadcasted_iota(jnp.int32, (nn, ee), 0) == dst_ref[...]
               ).astype(_F32)
        agg = jnp.dot(oht, acc_ref[...], preferred_element_type=_F32)
        out_ref[...] = _elu(
            agg + jnp.dot(x_ref[...], root_ref[...],
                          preferred_element_type=_F32) + bias_ref[...])


def _pool_body(h_ref, b_ref, w1_ref, b1_ref, w2_ref, b2_ref, w3_ref, b3_ref,
               out_ref, *, g):
    nn = h_ref.shape[0]
    ohb = (jax.lax.broadcasted_iota(jnp.int32, (g, nn), 0) == b_ref[...]
           ).astype(_F32)
    sums = jnp.dot(ohb, h_ref[...], preferred_element_type=_F32)
    cnt = jnp.sum(ohb, axis=1, keepdims=True)
    pooled = sums / jnp.maximum(cnt, 1.0)
    o = _elu(jnp.dot(pooled, w1_ref[...], preferred_element_type=_F32)
             + b1_ref[...])
    o = _elu(jnp.dot(o, w2_ref[...], preferred_element_type=_F32) + b2_ref[...])
    out_ref[...] = jnp.dot(o, w3_ref[...], preferred_element_type=_F32) \
        + b3_ref[...]


def _nnconv_layer(x, src_col, dst_row, h, wb, bb, root, bias, in_c, out_c,
                  hid, interpret=False):
    nn = x.shape[0]
    ee = src_col.shape[0]
    bbm = bb.reshape(in_c, out_c)
    bias2 = bias.reshape(1, out_c)
    if in_c == 37:
        body = functools.partial(_layer1_body, in_c=in_c, out_c=out_c)
        return pl.pallas_call(
            body,
            out_shape=jax.ShapeDtypeStruct((nn, out_c), _F32),
            interpret=interpret,
        )(x, src_col, dst_row, h, wb, bbm, root, bias2)
    ck = 8
    hc = hid // ck
    h3d = h.reshape(ee, hc, ck).transpose(1, 0, 2)
    wbf = wb.reshape(hid * in_c, out_c)
    body = functools.partial(_layer_body, ck=ck, in_c=in_c)
    return pl.pallas_call(
        body,
        grid=(hc,),
        out_shape=jax.ShapeDtypeStruct((nn, out_c), _F32),
        in_specs=[
            pl.BlockSpec((nn, in_c), lambda c: (0, 0)),
            pl.BlockSpec((ee, 1), lambda c: (0, 0)),
            pl.BlockSpec((1, ee), lambda c: (0, 0)),
            pl.BlockSpec((1, ee, ck), lambda c: (c, 0, 0)),
            pl.BlockSpec((ck * in_c, out_c), lambda c: (c, 0)),
            pl.BlockSpec((in_c, out_c), lambda c: (0, 0)),
            pl.BlockSpec((in_c, out_c), lambda c: (0, 0)),
            pl.BlockSpec((1, out_c), lambda c: (0, 0)),
        ],
        out_specs=pl.BlockSpec((nn, out_c), lambda c: (0, 0)),
        scratch_shapes=[
            pltpu.VMEM((ee, in_c), _F32),
            pltpu.VMEM((ee, out_c), _F32),
        ],
        compiler_params=pltpu.CompilerParams(
            dimension_semantics=("arbitrary",)),
        interpret=interpret,
    )(x, src_col, dst_row, h3d, wbf, bbm, root, bias2)


def _run(x, edge_index, edge_attr, batch, W1a, b1a, W1b, b1b, root1, bias1,
         W2a, b2a, W2b, b2b, root2, bias2, W3a, b3a, W3b, b3b, root3, bias3,
         Wfc1, bfc1, Wfc2, bfc2, Wfc3, bfc3, interpret=False):
    nn = x.shape[0]
    ee = edge_attr.shape[0]
    g = 64
    src_col = edge_index[0].reshape(ee, 1).astype(jnp.int32)
    dst_row = edge_index[1].reshape(1, ee).astype(jnp.int32)
    batch_row = batch.reshape(1, nn).astype(jnp.int32)

    ea_p = jnp.pad(edge_attr, ((0, 0), (0, 3)))
    w1a_p = jnp.pad(W1a, ((0, 3), (0, 0)))
    w2a_p = jnp.pad(W2a, ((0, 3), (0, 0)))
    w3a_p = jnp.pad(W3a, ((0, 3), (0, 0)))

    h1, h2, h3 = pl.pallas_call(
        _hidden_body,
        out_shape=(
            jax.ShapeDtypeStruct((ee, 512), _F32),
            jax.ShapeDtypeStruct((ee, 128), _F32),
            jax.ShapeDtypeStruct((ee, 128), _F32),
        ),
        interpret=interpret,
    )(ea_p, w1a_p, b1a.reshape(1, -1), w2a_p, b2a.reshape(1, -1),
      w3a_p, b3a.reshape(1, -1))

    x1 = _nnconv_layer(x, src_col, dst_row, h1, W1b, b1b, root1, bias1,
                       37, 128, 512, interpret)
    x2 = _nnconv_layer(x1, src_col, dst_row, h2, W2b, b2b, root2, bias2,
                       128, 256, 128, interpret)
    x3 = _nnconv_layer(x2, src_col, dst_row, h3, W3b, b3b, root3, bias3,
                       256, 256, 128, interpret)

    wfc3_p = jnp.pad(Wfc3, ((0, 0), (0, 127)))
    bfc3_p = jnp.pad(bfc3.reshape(1, 1), ((0, 0), (0, 127)))
    o = pl.pallas_call(
        functools.partial(_pool_body, g=g),
        out_shape=jax.ShapeDtypeStruct((g, 128), _F32),
        interpret=interpret,
    )(x3, batch_row, Wfc1, bfc1.reshape(1, -1), Wfc2, bfc2.reshape(1, -1),
      wfc3_p, bfc3_p)
    return o[:, 0]


def kernel(x, edge_index, edge_attr, batch, W1a, b1a, W1b, b1b, root1, bias1,
           W2a, b2a, W2b, b2b, root2, bias2, W3a, b3a, W3b, b3b, root3, bias3,
           Wfc1, bfc1, Wfc2, bfc2, Wfc3, bfc3):
    return _run(x, edge_index, edge_attr, batch, W1a, b1a, W1b, b1b, root1,
                bias1, W2a, b2a, W2b, b2b, root2, bias2, W3a, b3a, W3b, b3b,
                root3, bias3, Wfc1, bfc1, Wfc2, bfc2, Wfc3, bfc3)


# speedup vs baseline: 1.2798x; 1.2798x over previous
import functools

import jax
import jax.numpy as jnp
from jax.experimental import pallas as pl
from jax.experimental.pallas import tpu as pltpu

_F32 = jnp.float32


def _elu(v):
    return jnp.where(v > 0, v, jnp.exp(jnp.minimum(v, 0.0)) - 1.0)


def _hidden_body(ea_ref, w1_ref, b1_ref, w2_ref, b2_ref, w3_ref, b3_ref,
                 h1_ref, h2_ref, h3_ref):
    ea = ea_ref[...]
    h1_ref[...] = jnp.maximum(
        jnp.dot(ea, w1_ref[...], preferred_element_type=_F32) + b1_ref[...], 0.0)
    h2_ref[...] = jnp.maximum(
        jnp.dot(ea, w2_ref[...], preferred_element_type=_F32) + b2_ref[...], 0.0)
    h3_ref[...] = jnp.maximum(
        jnp.dot(ea, w3_ref[...], preferred_element_type=_F32) + b3_ref[...], 0.0)


def _layer1_body(x_ref, src_ref, dst_ref, h_ref, wb_ref, bbm_ref, root_ref,
                 bias_ref, out_ref, *, in_c, out_c):
    ee = src_ref.shape[0]
    nn = x_ref.shape[0]
    oh = (src_ref[...] == jax.lax.broadcasted_iota(jnp.int32, (ee, nn), 1)
          ).astype(_F32)
    xg = jnp.dot(oh, x_ref[...], preferred_element_type=_F32)
    acc = jnp.dot(xg, bbm_ref[...], preferred_element_type=_F32)
    h = h_ref[...]
    for i in range(in_c):
        t = jnp.dot(h, wb_ref[:, i * out_c:(i + 1) * out_c],
                    preferred_element_type=_F32)
        acc = acc + xg[:, i:i + 1] * t
    oht = (jax.lax.broadcasted_iota(jnp.int32, (nn, ee), 0) == dst_ref[...]
           ).astype(_F32)
    agg = jnp.dot(oht, acc, preferred_element_type=_F32)
    out_ref[...] = _elu(
        agg + jnp.dot(x_ref[...], root_ref[...], preferred_element_type=_F32)
        + bias_ref[...])


def _layer_body(x_ref, src_ref, dst_ref, h_ref, wb_ref, bbm_ref, root_ref,
                bias_ref, out_ref, xg_ref, acc_ref, *, ck, in_c):
    c = pl.program_id(0)
    ee = src_ref.shape[0]
    nn = x_ref.shape[0]

    @pl.when(c == 0)
    def _():
        oh = (src_ref[...] == jax.lax.broadcasted_iota(jnp.int32, (ee, nn), 1)
              ).astype(_F32)
        xg_ref[...] = jnp.dot(oh, x_ref[...], preferred_element_type=_F32)
        acc_ref[...] = jnp.dot(xg_ref[...], bbm_ref[...],
                               preferred_element_type=_F32)

    xg = xg_ref[...]
    hblk = h_ref[0]
    ts = [jnp.dot(hblk[:, j:j + 1] * xg, wb_ref[j * in_c:(j + 1) * in_c, :],
                  preferred_element_type=_F32) for j in range(ck)]
    while len(ts) > 1:
        ts = [ts[i] + ts[i + 1] for i in range(0, len(ts) - 1, 2)] \
            + ([ts[-1]] if len(ts) % 2 else [])
    acc_ref[...] = acc_ref[...] + ts[0]

    @pl.when(c == pl.num_programs(0) - 1)
    def _():
        oht = (jax.lax.broadcasted_iota(jnp.int32, (nn, ee), 0) == dst_ref[...]
               ).astype(_F32)
        agg = jnp.dot(oht, acc_ref[...], preferred_element_type=_F32)
        out_ref[...] = _elu(
            agg + jnp.dot(x_ref[...], root_ref[...],
                          preferred_element_type=_F32) + bias_ref[...])


def _pool_body(h_ref, b_ref, w1_ref, b1_ref, w2_ref, b2_ref, w3_ref, b3_ref,
               out_ref, *, g):
    nn = h_ref.shape[0]
    ohb = (jax.lax.broadcasted_iota(jnp.int32, (g, nn), 0) == b_ref[...]
           ).astype(_F32)
    sums = jnp.dot(ohb, h_ref[...], preferred_element_type=_F32)
    cnt = jnp.sum(ohb, axis=1, keepdims=True)
    pooled = sums / jnp.maximum(cnt, 1.0)
    o = _elu(jnp.dot(pooled, w1_ref[...], preferred_element_type=_F32)
             + b1_ref[...])
    o = _elu(jnp.dot(o, w2_ref[...], preferred_element_type=_F32) + b2_ref[...])
    out_ref[...] = jnp.dot(o, w3_ref[...], preferred_element_type=_F32) \
        + b3_ref[...]


def _nnconv_layer(x, src_col, dst_row, h, wb, bb, root, bias, in_c, out_c,
                  hid, interpret=False):
    nn = x.shape[0]
    ee = src_col.shape[0]
    bbm = bb.reshape(in_c, out_c)
    bias2 = bias.reshape(1, out_c)
    if in_c == 37:
        body = functools.partial(_layer1_body, in_c=in_c, out_c=out_c)
        return pl.pallas_call(
            body,
            out_shape=jax.ShapeDtypeStruct((nn, out_c), _F32),
            interpret=interpret,
        )(x, src_col, dst_row, h, wb, bbm, root, bias2)
    ck = 8
    hc = hid // ck
    h3d = h.reshape(ee, hc, ck).transpose(1, 0, 2)
    wbf = wb.reshape(hid * in_c, out_c)
    body = functools.partial(_layer_body, ck=ck, in_c=in_c)
    return pl.pallas_call(
        body,
        grid=(hc,),
        out_shape=jax.ShapeDtypeStruct((nn, out_c), _F32),
        in_specs=[
            pl.BlockSpec((nn, in_c), lambda c: (0, 0)),
            pl.BlockSpec((ee, 1), lambda c: (0, 0)),
            pl.BlockSpec((1, ee), lambda c: (0, 0)),
            pl.BlockSpec((1, ee, ck), lambda c: (c, 0, 0)),
            pl.BlockSpec((ck * in_c, out_c), lambda c: (c, 0)),
            pl.BlockSpec((in_c, out_c), lambda c: (0, 0)),
            pl.BlockSpec((in_c, out_c), lambda c: (0, 0)),
            pl.BlockSpec((1, out_c), lambda c: (0, 0)),
        ],
        out_specs=pl.BlockSpec((nn, out_c), lambda c: (0, 0)),
        scratch_shapes=[
            pltpu.VMEM((ee, in_c), _F32),
            pltpu.VMEM((ee, out_c), _F32),
        ],
        compiler_params=pltpu.CompilerParams(
            dimension_semantics=("arbitrary",)),
        interpret=interpret,
    )(x, src_col, dst_row, h3d, wbf, bbm, root, bias2)


def _run(x, edge_index, edge_attr, batch, W1a, b1a, W1b, b1b, root1, bias1,
         W2a, b2a, W2b, b2b, root2, bias2, W3a, b3a, W3b, b3b, root3, bias3,
         Wfc1, bfc1, Wfc2, bfc2, Wfc3, bfc3, interpret=False):
    nn = x.shape[0]
    ee = edge_attr.shape[0]
    g = 64
    src_col = edge_index[0].reshape(ee, 1).astype(jnp.int32)
    dst_row = edge_index[1].reshape(1, ee).astype(jnp.int32)
    batch_row = batch.reshape(1, nn).astype(jnp.int32)

    ea_p = jnp.pad(edge_attr, ((0, 0), (0, 3)))
    w1a_p = jnp.pad(W1a, ((0, 3), (0, 0)))
    w2a_p = jnp.pad(W2a, ((0, 3), (0, 0)))
    w3a_p = jnp.pad(W3a, ((0, 3), (0, 0)))

    h1, h2, h3 = pl.pallas_call(
        _hidden_body,
        out_shape=(
            jax.ShapeDtypeStruct((ee, 512), _F32),
            jax.ShapeDtypeStruct((ee, 128), _F32),
            jax.ShapeDtypeStruct((ee, 128), _F32),
        ),
        interpret=interpret,
    )(ea_p, w1a_p, b1a.reshape(1, -1), w2a_p, b2a.reshape(1, -1),
      w3a_p, b3a.reshape(1, -1))

    x1 = _nnconv_layer(x, src_col, dst_row, h1, W1b, b1b, root1, bias1,
                       37, 128, 512, interpret)
    x2 = _nnconv_layer(x1, src_col, dst_row, h2, W2b, b2b, root2, bias2,
                       128, 256, 128, interpret)
    x3 = _nnconv_layer(x2, src_col, dst_row, h3, W3b, b3b, root3, bias3,
                       256, 256, 128, interpret)

    wfc3_p = jnp.pad(Wfc3, ((0, 0), (0, 127)))
    bfc3_p = jnp.pad(bfc3.reshape(1, 1), ((0, 0), (0, 127)))
    o = pl.pallas_call(
        functools.partial(_pool_body, g=g),
        out_shape=jax.ShapeDtypeStruct((g, 128), _F32),
        interpret=interpret,
    )(x3, batch_row, Wfc1, bfc1.reshape(1, -1), Wfc2, bfc2.reshape(1, -1),
      wfc3_p, bfc3_p)
    return o[:, 0]


def kernel(x, edge_index, edge_attr, batch, W1a, b1a, W1b, b1b, root1, bias1,
           W2a, b2a, W2b, b2b, root2, bias2, W3a, b3a, W3b, b3b, root3, bias3,
           Wfc1, bfc1, Wfc2, bfc2, Wfc3, bfc3):
    return _run(x, edge_index, edge_attr, batch, W1a, b1a, W1b, b1b, root1,
                bias1, W2a, b2a, W2b, b2b, root2, bias2, W3a, b3a, W3b, b3b,
                root3, bias3, Wfc1, bfc1, Wfc2, bfc2, Wfc3, bfc3)
